# baseline (speedup 1.0000x reference)
import jax
import jax.numpy as jnp
from jax.experimental import pallas as pl
from jax.experimental.pallas import tpu as pltpu

_TOKEN_BLOCK = 2048


def _router_body(x_ref, w1_ref, b1_ref, w2t_ref, b2_ref, out_ref):
    xb = x_ref[...].astype(jnp.bfloat16)
    w1b = w1_ref[...].astype(jnp.bfloat16)
    w2b = w2t_ref[...].astype(jnp.bfloat16).T
    h = jnp.dot(xb, w1b, preferred_element_type=jnp.float32)
    h = jnp.maximum(h + b1_ref[...], 0.0)
    logits = jnp.dot(h.astype(jnp.bfloat16), w2b,
                     preferred_element_type=jnp.float32)
    lt = logits.T + b2_ref[...]
    m = jnp.max(lt, axis=0, keepdims=True)
    e = jnp.exp(lt - m)
    out_ref[...] = e / jnp.sum(e, axis=0, keepdims=True)


@jax.jit
def kernel(x, W1, b1, W2, b2):
    n_tokens, d_model = x.shape
    n_experts = W2.shape[1]
    bt = _TOKEN_BLOCK
    out_t = pl.pallas_call(
        _router_body,
        grid=(n_tokens // bt,),
        in_specs=[
            pl.BlockSpec((bt, d_model), lambda i: (i, 0)),
            pl.BlockSpec((d_model, d_model), lambda i: (0, 0)),
            pl.BlockSpec((1, d_model), lambda i: (0, 0)),
            pl.BlockSpec((n_experts, d_model), lambda i: (0, 0)),
            pl.BlockSpec((n_experts, 1), lambda i: (0, 0)),
        ],
        out_specs=pl.BlockSpec((n_experts, bt), lambda i: (0, i)),
        out_shape=jax.ShapeDtypeStruct((n_experts, n_tokens), jnp.float32),
        compiler_params=pltpu.CompilerParams(
            dimension_semantics=("parallel",),
        ),
    )(x, W1, b1.reshape(1, d_model), W2.T, b2.reshape(n_experts, 1))
    return out_t.T


# speedup vs baseline: 1.0329x; 1.0329x over previous
import jax
import jax.numpy as jnp
from jax.experimental import pallas as pl
from jax.experimental.pallas import tpu as pltpu

_TOKEN_BLOCK = 2048


def _router_body(x_ref, w1_ref, b1_ref, w2t_ref, b2_ref, out_ref):
    xb = x_ref[...].astype(jnp.bfloat16)
    w1b = w1_ref[...].astype(jnp.bfloat16)
    w2tb = w2t_ref[...].astype(jnp.bfloat16)
    h = jnp.dot(xb, w1b, preferred_element_type=jnp.float32)
    h = jnp.maximum(h + b1_ref[...], 0.0)
    lt = jax.lax.dot_general(w2tb, h.astype(jnp.bfloat16),
                             (((1,), (1,)), ((), ())),
                             preferred_element_type=jnp.float32)
    lt = lt + b2_ref[...]
    m = jnp.max(lt, axis=0, keepdims=True)
    e = jnp.exp(lt - m)
    out_ref[...] = e / jnp.sum(e, axis=0, keepdims=True)


@jax.jit
def kernel(x, W1, b1, W2, b2):
    n_tokens, d_model = x.shape
    n_experts = W2.shape[1]
    bt = _TOKEN_BLOCK
    out_t = pl.pallas_call(
        _router_body,
        grid=(n_tokens // bt,),
        in_specs=[
            pl.BlockSpec((bt, d_model), lambda i: (i, 0)),
            pl.BlockSpec((d_model, d_model), lambda i: (0, 0)),
            pl.BlockSpec((1, d_model), lambda i: (0, 0)),
            pl.BlockSpec((n_experts, d_model), lambda i: (0, 0)),
            pl.BlockSpec((n_experts, 1), lambda i: (0, 0)),
        ],
        out_specs=pl.BlockSpec((n_experts, bt), lambda i: (0, i)),
        out_shape=jax.ShapeDtypeStruct((n_experts, n_tokens), jnp.float32),
        compiler_params=pltpu.CompilerParams(
            dimension_semantics=("parallel",),
        ),
    )(x, W1, b1.reshape(1, d_model), W2.T, b2.reshape(n_experts, 1))
    return out_t.T


# speedup vs baseline: 1.0387x; 1.0055x over previous
import jax
import jax.numpy as jnp
from jax.experimental import pallas as pl
from jax.experimental.pallas import tpu as pltpu

_TOKEN_BLOCK = 2048


def _router_body(x_ref, w1_ref, b1_ref, w2t_ref, b2_ref, out_ref):
    xb = x_ref[...].astype(jnp.bfloat16)
    w1b = w1_ref[...].astype(jnp.bfloat16)
    w2tb = w2t_ref[...].astype(jnp.bfloat16)
    ht = jax.lax.dot_general(w1b, xb, (((0,), (1,)), ((), ())),
                             preferred_element_type=jnp.float32)
    ht = jnp.maximum(ht + b1_ref[...], 0.0)
    lt = jnp.dot(w2tb, ht.astype(jnp.bfloat16),
                 preferred_element_type=jnp.float32)
    lt = lt + b2_ref[...]
    m = jnp.max(lt, axis=0, keepdims=True)
    e = jnp.exp(lt - m)
    out_ref[...] = e / jnp.sum(e, axis=0, keepdims=True)


@jax.jit
def kernel(x, W1, b1, W2, b2):
    n_tokens, d_model = x.shape
    n_experts = W2.shape[1]
    bt = _TOKEN_BLOCK
    out_t = pl.pallas_call(
        _router_body,
        grid=(n_tokens // bt,),
        in_specs=[
            pl.BlockSpec((bt, d_model), lambda i: (i, 0)),
            pl.BlockSpec((d_model, d_model), lambda i: (0, 0)),
            pl.BlockSpec((d_model, 1), lambda i: (0, 0)),
            pl.BlockSpec((n_experts, d_model), lambda i: (0, 0)),
            pl.BlockSpec((n_experts, 1), lambda i: (0, 0)),
        ],
        out_specs=pl.BlockSpec((n_experts, bt), lambda i: (0, i)),
        out_shape=jax.ShapeDtypeStruct((n_experts, n_tokens), jnp.float32),
        compiler_params=pltpu.CompilerParams(
            dimension_semantics=("parallel",),
        ),
    )(x, W1, b1.reshape(d_model, 1), W2.T, b2.reshape(n_experts, 1))
    return out_t.T
